# Initial kernel scaffold; baseline (speedup 1.0000x reference)
import jax
import jax.numpy as jnp
from jax.experimental import pallas as pl


def kernel(X, edge_index, W_l, b_l, W_r):
    raise NotImplementedError("write your pallas kernel here")


# speedup vs baseline: 1.4281x; 1.4281x over previous
import functools

import jax
import jax.numpy as jnp
from jax import lax
from jax.experimental import pallas as pl
from jax.experimental.pallas import tpu as pltpu
from jax.experimental.pallas import tpu_sc as plsc

N_NODES = 10000
N_EDGES = 320000
D = 128
NC = 2
NS = 16
NW = NC * NS
R = 320
NPAD = NW * R
C = 3200
NEG = float("-inf")


def _body(src_h, dst_h, x_h, wl_h, wr_h, out_h,
          dstb, srcb, qsrc, qld, rows, acc, xrows, wlv, wrv, outv, sem):
    wid = lax.axis_index("s") * NC + lax.axis_index("c")
    lo = wid * R
    lov = jnp.full((16,), lo, jnp.int32)
    hiv = lov + R
    iota = lax.iota(jnp.int32, 16)

    def init_row(r, carry):
        for k in range(D // 16):
            acc[r, pl.ds(16 * k, 16)] = jnp.full((16,), NEG, jnp.float32)
        return carry
    lax.fori_loop(0, R + 1, init_row, 0)

    def chunk_body(c, carry):
        off = c * C
        pltpu.sync_copy(dst_h.at[pl.ds(off, C)], dstb)
        pltpu.sync_copy(src_h.at[pl.ds(off, C)], srcb)

        def scan_body(i, cursor):
            d = dstb[pl.ds(i * 16, 16)]
            s = srcb[pl.ds(i * 16, 16)]
            m = (d >= lov) & (d < hiv)
            mi = m.astype(jnp.int32)
            pos = cursor + plsc.cumsum(mi) - mi
            plsc.store_scatter(qsrc, [pos], s, mask=m)
            plsc.store_scatter(qld, [pos], d - lov, mask=m)
            return cursor + plsc.all_reduce_population_count(m)
        cursor = lax.fori_loop(0, C // 16, scan_body,
                               jnp.zeros((16,), jnp.int32))
        plsc.store_scatter(qsrc, [cursor + iota], jnp.zeros((16,), jnp.int32))
        plsc.store_scatter(qld, [cursor + iota], jnp.full((16,), R, jnp.int32))
        cnt = jnp.max(cursor)
        ngroups = lax.shift_right_logical(cnt + 15, 4)

        def group_body(g, carry):
            pltpu.async_copy(x_h.at[qsrc.at[pl.ds(g * 16, 16)]], rows,
                             sem).wait()
            ldv = qld[pl.ds(g * 16, 16)]
            for j in range(16):
                ld = ldv[j]
                for k in range(D // 16):
                    sl = pl.ds(16 * k, 16)
                    acc[ld, sl] = jnp.maximum(acc[ld, sl], rows[j, sl])
            return carry
        lax.fori_loop(0, ngroups, group_body, 0)
        return carry
    lax.fori_loop(0, N_EDGES // C, chunk_body, 0)

    pltpu.sync_copy(x_h.at[pl.ds(lo, R)], xrows)
    pltpu.sync_copy(wl_h, wlv)
    pltpu.sync_copy(wr_h, wrv)
    negv = jnp.full((16,), NEG, jnp.float32)

    def fin_body(r, carry):
        t = jnp.zeros((16,), jnp.float32)
        for k in range(D // 16):
            sl = pl.ds(16 * k, 16)
            a = acc[r, sl]
            a = jnp.where(a == negv, jnp.float32(0.0), a)
            t = t + a * wlv[sl] + xrows[r, sl] * wrv[sl]
        s = jnp.sum(t)
        plsc.store_scatter(outv, [jnp.full((16,), r, jnp.int32)],
                           jnp.full((16,), s, jnp.float32),
                           mask=lax.iota(jnp.int32, 16) == 0)
        return carry
    lax.fori_loop(0, R, fin_body, 0)
    pltpu.sync_copy(outv, out_h.at[pl.ds(lo, R)])


@jax.jit
def _sc_call(src, dst, xp, wl, wr):
    mesh = plsc.VectorSubcoreMesh(core_axis_name="c", subcore_axis_name="s",
                                  num_cores=NC, num_subcores=NS)
    return pl.kernel(
        _body,
        out_type=jax.ShapeDtypeStruct((NPAD,), jnp.float32),
        mesh=mesh,
        compiler_params=pltpu.CompilerParams(needs_layout_passes=False),
        scratch_types=[
            pltpu.VMEM((C,), jnp.int32),
            pltpu.VMEM((C,), jnp.int32),
            pltpu.VMEM((C + 16,), jnp.int32),
            pltpu.VMEM((C + 16,), jnp.int32),
            pltpu.VMEM((16, D), jnp.float32),
            pltpu.VMEM((R + 1, D), jnp.float32),
            pltpu.VMEM((R, D), jnp.float32),
            pltpu.VMEM((D,), jnp.float32),
            pltpu.VMEM((D,), jnp.float32),
            pltpu.VMEM((R,), jnp.float32),
            pltpu.SemaphoreType.DMA,
        ],
    )(src, dst, xp, wl, wr)


def kernel(X, edge_index, W_l, b_l, W_r):
    ei = edge_index.astype(jnp.int32)
    src = ei[0]
    dst = ei[1]
    xp = jnp.pad(X, ((0, NPAD - N_NODES), (0, 0)))
    out = _sc_call(src, dst, xp, W_l.reshape(-1), W_r.reshape(-1))
    return out[:N_NODES, None] + b_l[None, :]
